# baseline (speedup 1.0000x reference)
import functools

import jax
import jax.numpy as jnp
from jax import lax
from jax.experimental import pallas as pl
from jax.experimental.pallas import tpu as pltpu
from jax.experimental.pallas import tpu_sc as plsc

VOCAB = 1000000
EMBED_DIM = 16
BATCH = 16384

_INFO = plsc.get_sparse_core_info()
_NC = _INFO.num_cores
_NS = _INFO.num_subcores
_NW = _NC * _NS
_L = _INFO.num_lanes
_B_PER_W = BATCH // _NW
_W = 32
_N_WAVES = _B_PER_W // _W


@functools.partial(
    pl.kernel,
    mesh=plsc.VectorSubcoreMesh(core_axis_name="c", subcore_axis_name="s"),
    out_type=jax.ShapeDtypeStruct((BATCH * EMBED_DIM,), jnp.float32),
    scratch_types=[
        pltpu.VMEM((_B_PER_W,), jnp.int32),
        pltpu.VMEM((_W, EMBED_DIM, 128), jnp.float32),
        pltpu.VMEM((_B_PER_W * EMBED_DIM,), jnp.float32),
        pltpu.SemaphoreType.DMA,
    ],
    compiler_params=pltpu.CompilerParams(needs_layout_passes=False),
)
def _gather(idx_hbm, wt_hbm, out_hbm, idx_v, buf, out_v, sem):
    wid = lax.axis_index("s") * _NC + lax.axis_index("c")
    base = wid * _B_PER_W
    pltpu.sync_copy(idx_hbm.at[pl.ds(base, _B_PER_W)], idx_v)

    def wave(w, _):
        j0 = w * _W
        rs = []
        for g in range(_W // _L):
            vg = idx_v[pl.ds(j0 + g * _L, _L)]
            rs.extend(vg[i] for i in range(_L))
        for jj in range(_W):
            t = pl.multiple_of((rs[jj] >> 7) << 7, 128)
            pltpu.async_copy(
                wt_hbm.at[:, pl.ds(t, 128)], buf.at[jj], sem
            )
        for jj in range(_W):
            pltpu.make_async_copy(
                wt_hbm.at[:, pl.ds(0, 128)], buf.at[jj], sem
            ).wait()
        for jj in range(_W):
            lane = jnp.full((_L,), rs[jj] & 127, jnp.int32)
            slot = jnp.full((_L,), jj, jnp.int32)
            vals = plsc.load_gather(
                buf, [slot, lax.iota(jnp.int32, _L), lane]
            )
            out_v[pl.ds((j0 + jj) * EMBED_DIM, EMBED_DIM)] = vals
        return _

    lax.fori_loop(0, _N_WAVES, wave, 0)
    pltpu.sync_copy(out_v, out_hbm.at[pl.ds(base * EMBED_DIM,
                                            _B_PER_W * EMBED_DIM)])


def kernel(indices, weight):
    return _gather(indices.astype(jnp.int32), weight.T)


# speedup vs baseline: 1.0467x; 1.0467x over previous
import functools

import jax
import jax.numpy as jnp
from jax import lax
from jax.experimental import pallas as pl
from jax.experimental.pallas import tpu as pltpu
from jax.experimental.pallas import tpu_sc as plsc

VOCAB = 1000000
EMBED_DIM = 16
BATCH = 16384

_INFO = plsc.get_sparse_core_info()
_NC = _INFO.num_cores
_NS = _INFO.num_subcores
_NW = _NC * _NS
_L = _INFO.num_lanes
_B_PER_W = BATCH // _NW
_W = 16
_N_WAVES = _B_PER_W // _W


@functools.partial(
    pl.kernel,
    mesh=plsc.VectorSubcoreMesh(core_axis_name="c", subcore_axis_name="s"),
    out_type=jax.ShapeDtypeStruct((BATCH * EMBED_DIM,), jnp.float32),
    scratch_types=[
        pltpu.VMEM((_B_PER_W,), jnp.int32),
        pltpu.VMEM((2, _W, EMBED_DIM, 128), jnp.float32),
        pltpu.VMEM((_B_PER_W * EMBED_DIM,), jnp.float32),
        pltpu.SemaphoreType.DMA,
        pltpu.SemaphoreType.DMA,
    ],
    compiler_params=pltpu.CompilerParams(needs_layout_passes=False),
)
def _gather(idx_hbm, wt_hbm, out_hbm, idx_v, buf, out_v, sem_a, sem_b):
    wid = lax.axis_index("s") * _NC + lax.axis_index("c")
    base = wid * _B_PER_W
    pltpu.sync_copy(idx_hbm.at[pl.ds(base, _B_PER_W)], idx_v)
    sems = (sem_a, sem_b)

    def load_scalars(w):
        j0 = w * _W
        vg = idx_v[pl.ds(j0, _L)]
        return [vg[i] for i in range(_L)]

    def fire(rs, half):
        for jj in range(_W):
            t = pl.multiple_of((rs[jj] >> 7) << 7, 128)
            pltpu.async_copy(
                wt_hbm.at[:, pl.ds(t, 128)], buf.at[half, jj], sems[half]
            )

    def drain_extract(rs, w, half):
        j0 = w * _W
        for jj in range(_W):
            pltpu.make_async_copy(
                wt_hbm.at[:, pl.ds(0, 128)], buf.at[half, jj], sems[half]
            ).wait()
        for jj in range(_W):
            lane = jnp.full((_L,), rs[jj] & 127, jnp.int32)
            half_v = jnp.full((_L,), half, jnp.int32)
            slot = jnp.full((_L,), jj, jnp.int32)
            vals = plsc.load_gather(
                buf, [half_v, slot, lax.iota(jnp.int32, _L), lane]
            )
            out_v[pl.ds((j0 + jj) * EMBED_DIM, EMBED_DIM)] = vals

    rs0 = load_scalars(0)
    fire(rs0, 0)

    def body(k, _):
        wa = k * 2
        rs_a = load_scalars(wa)
        rs_b = load_scalars(wa + 1)
        fire(rs_b, 1)
        drain_extract(rs_a, wa, 0)

        @pl.when(k < _N_WAVES // 2 - 1)
        def _fire_next():
            fire(load_scalars(wa + 2), 0)

        drain_extract(rs_b, wa + 1, 1)
        return _

    lax.fori_loop(0, _N_WAVES // 2, body, 0)
    pltpu.sync_copy(out_v, out_hbm.at[pl.ds(base * EMBED_DIM,
                                            _B_PER_W * EMBED_DIM)])


def kernel(indices, weight):
    return _gather(indices.astype(jnp.int32), weight.T)


# speedup vs baseline: 1.0694x; 1.0217x over previous
import functools

import jax
import jax.numpy as jnp
from jax import lax
from jax.experimental import pallas as pl
from jax.experimental.pallas import tpu as pltpu
from jax.experimental.pallas import tpu_sc as plsc

VOCAB = 1000000
EMBED_DIM = 16
BATCH = 16384

_INFO = plsc.get_sparse_core_info()
_NC = _INFO.num_cores
_NS = _INFO.num_subcores
_NW = _NC * _NS
_L = _INFO.num_lanes
_B_PER_W = BATCH // _NW
_W = 16
_N_WAVES = _B_PER_W // _W
_HALF_COLS = _W * 128


@functools.partial(
    pl.kernel,
    mesh=plsc.VectorSubcoreMesh(core_axis_name="c", subcore_axis_name="s"),
    out_type=jax.ShapeDtypeStruct((BATCH * EMBED_DIM,), jnp.float32),
    scratch_types=[
        pltpu.VMEM((_B_PER_W,), jnp.int32),
        pltpu.VMEM((EMBED_DIM, 2 * _HALF_COLS), jnp.float32),
        pltpu.VMEM((_B_PER_W * EMBED_DIM,), jnp.float32),
        pltpu.SemaphoreType.DMA,
        pltpu.SemaphoreType.DMA,
    ],
    compiler_params=pltpu.CompilerParams(needs_layout_passes=False),
)
def _gather(idx_hbm, wt_hbm, out_hbm, idx_v, buf, out_v, sem_a, sem_b):
    wid = lax.axis_index("s") * _NC + lax.axis_index("c")
    base = wid * _B_PER_W
    pltpu.sync_copy(idx_hbm.at[pl.ds(base, _B_PER_W)], idx_v)
    sems = (sem_a, sem_b)
    lane16 = lax.iota(jnp.int32, _L)

    def fire(w, half):
        vg = idx_v[pl.ds(w * _W, _L)]
        rs = [vg[i] for i in range(_L)]
        for jj in range(_W):
            t = pl.multiple_of((rs[jj] >> 7) << 7, 128)
            pltpu.async_copy(
                wt_hbm.at[:, pl.ds(t, 128)],
                buf.at[:, pl.ds(half * _HALF_COLS + jj * 128, 128)],
                sems[half],
            )

    def drain_extract(w, half):
        pltpu.make_async_copy(
            wt_hbm.at[:, pl.ds(0, _HALF_COLS)],
            buf.at[:, pl.ds(half * _HALF_COLS, _HALF_COLS)],
            sems[half],
        ).wait()
        j0 = w * _W
        idxs = idx_v[pl.ds(j0, _L)]
        cols = half * _HALF_COLS + lane16 * 128 + (idxs & 127)
        pos_base = (j0 + lane16) * EMBED_DIM
        for c in range(EMBED_DIM):
            vals = plsc.load_gather(
                buf, [jnp.full((_L,), c, jnp.int32), cols]
            )
            plsc.store_scatter(out_v, [pos_base + c], vals)

    fire(0, 0)

    def body(k, _):
        wa = k * 2
        fire(wa + 1, 1)
        drain_extract(wa, 0)

        @pl.when(k < _N_WAVES // 2 - 1)
        def _fire_next():
            fire(wa + 2, 0)

        drain_extract(wa + 1, 1)
        return _

    lax.fori_loop(0, _N_WAVES // 2, body, 0)
    pltpu.sync_copy(out_v, out_hbm.at[pl.ds(base * EMBED_DIM,
                                            _B_PER_W * EMBED_DIM)])


def kernel(indices, weight):
    return _gather(indices.astype(jnp.int32), weight.T)
